# baseline (speedup 1.0000x reference)
import jax
import jax.numpy as jnp
from jax import lax
from jax.experimental import pallas as pl
from jax.experimental.pallas import tpu as pltpu

_EXPERTS = 64
_TOPK = 2


def _route_block(w, h, gates_ref, idx_ref, acc_sum, acc_pos):
    logits = lax.dot_general(
        w, h,
        dimension_numbers=(((1,), (1,)), ((), ())),
        preferred_element_type=jnp.float32)
    e, tm = logits.shape
    row = lax.broadcasted_iota(jnp.int32, (e, tm), 0)

    m1 = jnp.max(logits, axis=0, keepdims=True)
    i1 = jnp.min(jnp.where(logits == m1, row, e), axis=0, keepdims=True)
    masked = jnp.where(row == i1, jnp.float32(-jnp.inf), logits)
    m2 = jnp.max(masked, axis=0, keepdims=True)
    i2 = jnp.min(jnp.where(masked == m2, row, e), axis=0, keepdims=True)

    e2 = jnp.exp(m2 - m1)
    denom = 1.0 + e2
    gates_ref[...] = jnp.concatenate([1.0 / denom, e2 / denom], axis=0)
    idx_ref[...] = jnp.concatenate([i1, i2], axis=0)

    p = jnp.exp(logits - m1)
    pn = p / jnp.sum(p, axis=0, keepdims=True)
    acc_sum[...] += pn
    acc_pos[...] += (pn > 0).astype(jnp.float32)


def _router_body(w_ref, ha_ref, hb_ref, ga_ref, ia_ref, gb_ref, ib_ref,
                 loss_ref, acc_sum, acc_pos):
    pid = pl.program_id(0)
    nprog = pl.num_programs(0)

    @pl.when(pid == 0)
    def _init():
        acc_sum[...] = jnp.zeros_like(acc_sum)
        acc_pos[...] = jnp.zeros_like(acc_pos)

    w = w_ref[...]
    _route_block(w, ha_ref[0], ga_ref, ia_ref, acc_sum, acc_pos)
    _route_block(w, hb_ref[0], gb_ref, ib_ref, acc_sum, acc_pos)

    @pl.when(pid == nprog - 1)
    def _fin():
        e, tm = acc_sum.shape
        n_tok = jnp.float32(nprog * tm * 2)
        s_e = jnp.sum(acc_sum[...], axis=1)
        c_e = jnp.sum(acc_pos[...], axis=1)
        loss = (jnp.float32(e) / (n_tok * n_tok)) * jnp.sum(
            s_e * c_e, keepdims=True)
        loss_ref[...] = loss.reshape(1, 1)


def _run(hidden_flat, w, tm, interpret=False):
    n, h = hidden_flat.shape
    e = w.shape[0]
    half = n // 2
    h3 = hidden_flat.reshape(2, half, h)
    grid = (half // tm,)
    return pl.pallas_call(
        _router_body,
        grid=grid,
        in_specs=[
            pl.BlockSpec((e, h), lambda i: (0, 0)),
            pl.BlockSpec((1, tm, h), lambda i: (0, i, 0)),
            pl.BlockSpec((1, tm, h), lambda i: (1, i, 0)),
        ],
        out_specs=[
            pl.BlockSpec((_TOPK, tm), lambda i: (0, i)),
            pl.BlockSpec((_TOPK, tm), lambda i: (0, i)),
            pl.BlockSpec((_TOPK, tm), lambda i: (0, i)),
            pl.BlockSpec((_TOPK, tm), lambda i: (0, i)),
            pl.BlockSpec((1, 1), lambda i: (0, 0)),
        ],
        out_shape=[
            jax.ShapeDtypeStruct((_TOPK, half), jnp.float32),
            jax.ShapeDtypeStruct((_TOPK, half), jnp.int32),
            jax.ShapeDtypeStruct((_TOPK, half), jnp.float32),
            jax.ShapeDtypeStruct((_TOPK, half), jnp.int32),
            jax.ShapeDtypeStruct((1, 1), jnp.float32),
        ],
        scratch_shapes=[
            pltpu.VMEM((e, tm), jnp.float32),
            pltpu.VMEM((e, tm), jnp.float32),
        ],
        compiler_params=pltpu.CompilerParams(
            dimension_semantics=("arbitrary",)),
        interpret=interpret,
    )(w, h3, h3)


def kernel(hidden_states, W):
    b, s, h = hidden_states.shape
    hf = hidden_states.reshape(b * s, h)
    ga, ia, gb, ib, loss = _run(hf, W, tm=4096)
    gates = jnp.concatenate([ga, gb], axis=1).T.reshape(b, s, _TOPK)
    idx = jnp.concatenate([ia, ib], axis=1).T.reshape(b, s, _TOPK)
    return (gates, idx, loss[0, 0])


# speedup vs baseline: 1.1203x; 1.1203x over previous
import jax
import jax.numpy as jnp
from jax import lax
from jax.experimental import pallas as pl
from jax.experimental.pallas import tpu as pltpu

_EXPERTS = 64
_TOPK = 2


def _route_block(w, h, gates_ref, idx_ref, acc_sum, acc_pos):
    logits = lax.dot_general(
        w, h,
        dimension_numbers=(((1,), (1,)), ((), ())),
        preferred_element_type=jnp.float32)
    e, tm = logits.shape
    row = lax.broadcasted_iota(jnp.int32, (e, tm), 0)

    m1 = jnp.max(logits, axis=0, keepdims=True)
    i1 = jnp.min(jnp.where(logits == m1, row, e), axis=0, keepdims=True)
    masked = jnp.where(row == i1, jnp.float32(-jnp.inf), logits)
    m2 = jnp.max(masked, axis=0, keepdims=True)
    i2 = jnp.min(jnp.where(masked == m2, row, e), axis=0, keepdims=True)

    e2 = jnp.exp(m2 - m1)
    denom = 1.0 + e2
    gates_ref[...] = jnp.concatenate([1.0 / denom, e2 / denom], axis=0)
    idx_ref[...] = jnp.concatenate([i1, i2], axis=0)

    p = jnp.exp(logits - m1)
    pn = p / jnp.sum(p, axis=0, keepdims=True)
    acc_sum[...] += pn
    acc_pos[...] += (pn > 0).astype(jnp.float32)


def _router_body(w_ref, ha_ref, hb_ref, ga_ref, ia_ref, gb_ref, ib_ref,
                 loss_ref, acc_sum, acc_pos):
    pid = pl.program_id(0)
    nprog = pl.num_programs(0)

    @pl.when(pid == 0)
    def _init():
        acc_sum[...] = jnp.zeros_like(acc_sum)
        acc_pos[...] = jnp.zeros_like(acc_pos)

    w = w_ref[...]
    _route_block(w, ha_ref[0], ga_ref, ia_ref, acc_sum, acc_pos)
    _route_block(w, hb_ref[0], gb_ref, ib_ref, acc_sum, acc_pos)

    @pl.when(pid == nprog - 1)
    def _fin():
        e, tm = acc_sum.shape
        n_tok = jnp.float32(nprog * tm * 2)
        s_e = jnp.sum(acc_sum[...], axis=1)
        c_e = jnp.sum(acc_pos[...], axis=1)
        loss = (jnp.float32(e) / (n_tok * n_tok)) * jnp.sum(
            s_e * c_e, keepdims=True)
        loss_ref[...] = loss.reshape(1, 1)


def _run(hidden_flat, w, tm, interpret=False):
    n, h = hidden_flat.shape
    e = w.shape[0]
    half = n // 2
    h3 = hidden_flat.reshape(2, half, h)
    grid = (half // tm,)
    return pl.pallas_call(
        _router_body,
        grid=grid,
        in_specs=[
            pl.BlockSpec((e, h), lambda i: (0, 0)),
            pl.BlockSpec((1, tm, h), lambda i: (0, i, 0)),
            pl.BlockSpec((1, tm, h), lambda i: (1, i, 0)),
        ],
        out_specs=[
            pl.BlockSpec((_TOPK, tm), lambda i: (0, i)),
            pl.BlockSpec((_TOPK, tm), lambda i: (0, i)),
            pl.BlockSpec((_TOPK, tm), lambda i: (0, i)),
            pl.BlockSpec((_TOPK, tm), lambda i: (0, i)),
            pl.BlockSpec((1, 1), lambda i: (0, 0)),
        ],
        out_shape=[
            jax.ShapeDtypeStruct((_TOPK, half), jnp.float32),
            jax.ShapeDtypeStruct((_TOPK, half), jnp.int32),
            jax.ShapeDtypeStruct((_TOPK, half), jnp.float32),
            jax.ShapeDtypeStruct((_TOPK, half), jnp.int32),
            jax.ShapeDtypeStruct((1, 1), jnp.float32),
        ],
        scratch_shapes=[
            pltpu.VMEM((e, tm), jnp.float32),
            pltpu.VMEM((e, tm), jnp.float32),
        ],
        compiler_params=pltpu.CompilerParams(
            dimension_semantics=("arbitrary",)),
        interpret=interpret,
    )(w, h3, h3)


def kernel(hidden_states, W):
    b, s, h = hidden_states.shape
    hf = hidden_states.reshape(b * s, h)
    ga, ia, gb, ib, loss = _run(hf, W, tm=2048)
    gates = jnp.concatenate([ga, gb], axis=1).T.reshape(b, s, _TOPK)
    idx = jnp.concatenate([ia, ib], axis=1).T.reshape(b, s, _TOPK)
    return (gates, idx, loss[0, 0])


# speedup vs baseline: 1.3416x; 1.1975x over previous
import jax
import jax.numpy as jnp
from jax import lax
from jax.experimental import pallas as pl
from jax.experimental.pallas import tpu as pltpu

_EXPERTS = 64
_TOPK = 2


def _router_body(w_ref, h_ref, gates_ref, idx_ref, loss_ref, acc_sum, acc_pos):
    pid = pl.program_id(0)
    nprog = pl.num_programs(0)

    @pl.when(pid == 0)
    def _init():
        acc_sum[...] = jnp.zeros_like(acc_sum)
        acc_pos[...] = jnp.zeros_like(acc_pos)

    logits = lax.dot_general(
        w_ref[...], h_ref[...],
        dimension_numbers=(((1,), (1,)), ((), ())),
        preferred_element_type=jnp.float32)
    e, tm = logits.shape
    row = lax.broadcasted_iota(jnp.int32, (e, tm), 0)

    m1 = jnp.max(logits, axis=0, keepdims=True)
    i1 = jnp.min(jnp.where(logits == m1, row, e), axis=0, keepdims=True)
    masked = jnp.where(row == i1, jnp.float32(-jnp.inf), logits)
    m2 = jnp.max(masked, axis=0, keepdims=True)
    i2 = jnp.min(jnp.where(masked == m2, row, e), axis=0, keepdims=True)

    e2 = jnp.exp(m2 - m1)
    denom = 1.0 + e2
    gates_ref[...] = jnp.concatenate([1.0 / denom, e2 / denom], axis=0)
    idx_ref[...] = jnp.concatenate([i1, i2], axis=0)

    p = jnp.exp(logits - m1)
    pn = p / jnp.sum(p, axis=0, keepdims=True)
    acc_sum[...] += pn
    acc_pos[...] += (pn > 0).astype(jnp.float32)

    @pl.when(pid == nprog - 1)
    def _fin():
        n_tok = jnp.float32(nprog * tm)
        s_e = jnp.sum(acc_sum[...], axis=1)
        c_e = jnp.sum(acc_pos[...], axis=1)
        loss = (jnp.float32(e) / (n_tok * n_tok)) * jnp.sum(
            s_e * c_e, keepdims=True)
        loss_ref[...] = loss.reshape(1, 1)


def _run(hidden_flat, w, tm, interpret=False):
    n, h = hidden_flat.shape
    e = w.shape[0]
    grid = (n // tm,)
    return pl.pallas_call(
        _router_body,
        grid=grid,
        in_specs=[
            pl.BlockSpec((e, h), lambda i: (0, 0)),
            pl.BlockSpec((tm, h), lambda i: (i, 0)),
        ],
        out_specs=[
            pl.BlockSpec((_TOPK, tm), lambda i: (0, i)),
            pl.BlockSpec((_TOPK, tm), lambda i: (0, i)),
            pl.BlockSpec((1, 1), lambda i: (0, 0)),
        ],
        out_shape=[
            jax.ShapeDtypeStruct((_TOPK, n), jnp.float32),
            jax.ShapeDtypeStruct((_TOPK, n), jnp.int32),
            jax.ShapeDtypeStruct((1, 1), jnp.float32),
        ],
        scratch_shapes=[
            pltpu.VMEM((e, tm), jnp.float32),
            pltpu.VMEM((e, tm), jnp.float32),
        ],
        compiler_params=pltpu.CompilerParams(
            dimension_semantics=("arbitrary",)),
        interpret=interpret,
    )(w, hidden_flat)


def kernel(hidden_states, W):
    b, s, h = hidden_states.shape
    hf = hidden_states.reshape(b * s, h)
    gates_t, idx_t, loss = _run(hf, W, tm=4096)
    gates = gates_t.T.reshape(b, s, _TOPK)
    idx = idx_t.T.reshape(b, s, _TOPK)
    return (gates, idx, loss[0, 0])
